# baseline (speedup 1.0000x reference)
import functools

import jax
import jax.numpy as jnp
import numpy as np
from jax import lax
from jax.experimental import pallas as pl
from jax.experimental.pallas import tpu as pltpu
from jax.experimental.pallas import tpu_sc as plsc

H_BLK = 16
SC_NC = 2
SC_NS = 16
SC_CHUNK = 128


def _mm_t(x, w):
    return jax.lax.dot_general(x, w, (((1,), (1,)), ((), ())),
                               preferred_element_type=jnp.float32)


def _lane_cumsum(x):
    r, n = x.shape
    s = 1
    while s < n:
        shifted = jnp.concatenate(
            [jnp.zeros((r, s), dtype=x.dtype), x[:, : n - s]], axis=1)
        x = x + shifted
        s *= 2
    return x


def _leaky(x):
    return jnp.where(x >= 0, x, 0.01 * x)


def _stage_a(cls_ref, feats_ref, wh_ref, bh_ref, wt_ref, bt_ref,
             eh_ref, et_ref, idx_ref, *, k):
    cls = cls_ref[0]
    feats = feats_ref[0]
    h_dim, d = cls.shape
    t_dim = feats.shape[0]
    scale = d ** (-0.5)

    e_h = _mm_t(cls, wh_ref[...]) + bh_ref[...]
    e_t_cls = _mm_t(cls, wt_ref[...]) + bt_ref[...]
    e_t_feats = _mm_t(feats, wt_ref[...]) + bt_ref[...]
    eh_ref[0] = e_h
    et_ref[0, pl.ds(0, h_dim), :] = e_t_cls
    et_ref[0, pl.ds(h_dim, t_dim), :] = e_t_feats

    e_hs = e_h * scale
    attn = jnp.concatenate(
        [_mm_t(e_hs, e_t_cls), _mm_t(e_hs, e_t_feats)], axis=1)

    bits = jax.lax.bitcast_convert_type(attn, jnp.int32)
    skey = bits ^ (jax.lax.shift_right_arithmetic(bits, 31)
                   & jnp.int32(0x7FFFFFFF))
    sign = jnp.int32(np.int32(np.uint32(0x80000000)))
    prefix_u = jnp.zeros((h_dim, 1), dtype=jnp.int32)
    for b in range(31, -1, -1):
        bitc = jnp.int32(np.int32(np.uint32(1 << b)))
        cand_u = prefix_u | bitc
        cand_s = cand_u ^ sign
        cnt = jnp.sum((skey >= cand_s).astype(jnp.int32), axis=1,
                      keepdims=True)
        prefix_u = jnp.where(cnt >= k, cand_u, prefix_u)
    thr_s = prefix_u ^ sign

    mask_gt = skey > thr_s
    mask_eq = skey == thr_s
    cnt_gt = jnp.sum(mask_gt.astype(jnp.float32), axis=1, keepdims=True)
    need = jnp.float32(k) - cnt_gt
    rank_eq = _lane_cumsum(mask_eq.astype(jnp.float32))
    mask = mask_gt | (mask_eq & (rank_eq <= need))
    rank = _lane_cumsum(mask.astype(jnp.int32))
    rank_sel = jnp.where(mask, rank - 1, -1)

    n = h_dim + t_dim
    for blk in range(h_dim // H_BLK):
        h0 = blk * H_BLK
        rk3 = jnp.broadcast_to(rank_sel[h0:h0 + H_BLK, :][:, None, :],
                               (H_BLK, k, n))
        s_iota = jax.lax.broadcasted_iota(jnp.int32, (H_BLK, k, n), 1)
        j_iota = jax.lax.broadcasted_iota(jnp.int32, (H_BLK, k, n), 2)
        idx_blk = jnp.sum(jnp.where(rk3 == s_iota, j_iota, 0), axis=2)
        idx_ref[0, pl.ds(h0, H_BLK), :] = idx_blk


def _sc_gather(idx_hbm, table_hbm, nb_hbm, idx_v, rows_v, sem, *, ipw, chunk):
    wid = lax.axis_index("s") * SC_NC + lax.axis_index("c")
    base = wid * ipw
    pltpu.sync_copy(idx_hbm.at[pl.ds(base, ipw)], idx_v)

    def g_body(c, carry):
        pltpu.async_copy(table_hbm.at[idx_v.at[pl.ds(c * chunk, chunk)]],
                         rows_v, sem).wait()
        pltpu.sync_copy(rows_v, nb_hbm.at[pl.ds(base + c * chunk, chunk)])
        return carry
    lax.fori_loop(0, ipw // chunk, g_body, 0)


def _stage_c(cls_ref, eh_ref, nb_ref, w1_ref, b1_ref, w2_ref, b2_ref,
             gamma_ref, beta_ref, out_ref, enh_ref, *, k, n_blk):
    blk = pl.program_id(1)
    h_dim, d = cls_ref.shape[1:]
    scale = d ** (-0.5)

    h0 = blk * H_BLK
    nb3 = nb_ref[0, 0].reshape(H_BLK, k, d)
    e_h_blk = eh_ref[0, pl.ds(h0, H_BLK), :]
    e_hs_b = (e_h_blk * scale)[:, None, :]
    sel_logit = jnp.sum(nb3 * e_hs_b, axis=2, keepdims=True)
    m = jnp.max(sel_logit, axis=1, keepdims=True)
    ex = jnp.exp(sel_logit - m)
    pk = ex / jnp.sum(ex, axis=1, keepdims=True)

    arg = (2.0 - pk) * e_h_blk[:, None, :] + pk * nb3
    gate = jnp.tanh(arg)
    kaw = jnp.sum(nb3 * gate, axis=2, keepdims=True)
    km = jnp.max(kaw, axis=1, keepdims=True)
    kex = jnp.exp(kaw - km)
    ka_prob = kex / jnp.sum(kex, axis=1, keepdims=True)
    e_nh = jnp.sum(ka_prob * nb3, axis=1)
    enh_ref[pl.ds(h0, H_BLK), :] = e_nh

    @pl.when(blk == n_blk - 1)
    def _tail():
        cls = cls_ref[0]
        e_h = eh_ref[0]
        e_nh_all = enh_ref[...]
        sum_in = (e_h + e_nh_all) * 0.1 + cls
        bi_in = e_h * e_nh_all * 0.1 + cls
        s_emb = _leaky(_mm_t(sum_in, w1_ref[...]) + b1_ref[...])
        b_emb = _leaky(_mm_t(bi_in, w2_ref[...]) + b2_ref[...])
        emb = s_emb + b_emb
        mu = jnp.mean(emb, axis=-1, keepdims=True)
        var = jnp.mean((emb - mu) ** 2, axis=-1, keepdims=True)
        out = (emb - mu) / jnp.sqrt(var + 1e-5) * gamma_ref[...] + beta_ref[...]
        out_ref[0] = out


@jax.jit
def kernel(cls_tokens, feats, Wh, bh, Wt, bt, W1, b1, W2, b2, gamma, beta):
    b_dim, h_dim, d = cls_tokens.shape
    t_dim = feats.shape[1]
    n = h_dim + t_dim
    k = max(1, min(t_dim, int(0.5 * max(1, h_dim))))

    row = lambda v: v.reshape(1, d)
    full = lambda shape: pl.BlockSpec(shape, lambda *g: (0,) * len(shape))

    n_blk = h_dim // H_BLK
    n_rows = h_dim * k
    ipw = n_rows // (SC_NC * SC_NS)
    mesh = plsc.VectorSubcoreMesh(core_axis_name="c", subcore_axis_name="s")

    stage_a = pl.pallas_call(
        functools.partial(_stage_a, k=k),
        grid=(1,),
        in_specs=[
            pl.BlockSpec((1, h_dim, d), lambda b: (b, 0, 0)),
            pl.BlockSpec((1, t_dim, d), lambda b: (b, 0, 0)),
            full((d, d)), full((1, d)),
            full((d, d)), full((1, d)),
        ],
        out_specs=[
            pl.BlockSpec((1, h_dim, d), lambda b: (b, 0, 0)),
            pl.BlockSpec((1, n, d), lambda b: (b, 0, 0)),
            pl.BlockSpec((1, h_dim, k), lambda b: (b, 0, 0)),
        ],
        out_shape=[
            jax.ShapeDtypeStruct((1, h_dim, d), jnp.float32),
            jax.ShapeDtypeStruct((1, n, d), jnp.float32),
            jax.ShapeDtypeStruct((1, h_dim, k), jnp.int32),
        ],
    )

    sc_gather = pl.kernel(
        functools.partial(_sc_gather, ipw=ipw, chunk=SC_CHUNK),
        mesh=mesh,
        out_type=jax.ShapeDtypeStruct((n_rows, d), jnp.float32),
        scratch_types=[
            pltpu.VMEM((ipw,), jnp.int32),
            pltpu.VMEM((SC_CHUNK, d), jnp.float32),
            pltpu.SemaphoreType.DMA,
        ],
    )

    stage_c = pl.pallas_call(
        functools.partial(_stage_c, k=k, n_blk=n_blk),
        grid=(1, n_blk),
        in_specs=[
            pl.BlockSpec((1, h_dim, d), lambda b, j: (b, 0, 0)),
            pl.BlockSpec((1, h_dim, d), lambda b, j: (b, 0, 0)),
            pl.BlockSpec((1, 1, H_BLK * k, d), lambda b, j: (b, j, 0, 0)),
            full((d, d)), full((1, d)),
            full((d, d)), full((1, d)),
            full((1, d)), full((1, d)),
        ],
        out_specs=pl.BlockSpec((1, h_dim, d), lambda b, j: (b, 0, 0)),
        out_shape=jax.ShapeDtypeStruct((1, h_dim, d), jnp.float32),
        scratch_shapes=[
            pltpu.VMEM((h_dim, d), jnp.float32),
        ],
    )

    outs = []
    for b in range(b_dim):
        eh_b, et_b, idx_b = stage_a(
            cls_tokens[b:b + 1], feats[b:b + 1], Wh, row(bh), Wt, row(bt))
        nb_b = sc_gather(idx_b.reshape(n_rows), et_b.reshape(n, d))
        outs.append(stage_c(
            cls_tokens[b:b + 1], eh_b,
            nb_b.reshape(1, n_blk, H_BLK * k, d),
            W1, row(b1), W2, row(b2), row(gamma), row(beta)))
    return jnp.concatenate(outs, axis=0)


# speedup vs baseline: 1.0848x; 1.0848x over previous
import functools

import jax
import jax.numpy as jnp
import numpy as np
from jax import lax
from jax.experimental import pallas as pl
from jax.experimental.pallas import tpu as pltpu
from jax.experimental.pallas import tpu_sc as plsc

H_BLK = 16
SC_NC = 2
SC_NS = 16
SC_CHUNK = 128


def _mm_t(x, w):
    return jax.lax.dot_general(x, w, (((1,), (1,)), ((), ())),
                               preferred_element_type=jnp.float32)


def _lane_cumsum(x):
    r, n = x.shape
    s = 1
    while s < n:
        shifted = jnp.concatenate(
            [jnp.zeros((r, s), dtype=x.dtype), x[:, : n - s]], axis=1)
        x = x + shifted
        s *= 2
    return x


def _leaky(x):
    return jnp.where(x >= 0, x, 0.01 * x)


def _stage_a(cls_ref, feats_ref, wh_ref, bh_ref, wt_ref, bt_ref,
             eh_ref, et_ref, idx_ref, *, k):
    cls = cls_ref[0]
    feats = feats_ref[0]
    h_dim, d = cls.shape
    t_dim = feats.shape[0]
    scale = d ** (-0.5)

    e_h = _mm_t(cls, wh_ref[...]) + bh_ref[...]
    e_t_cls = _mm_t(cls, wt_ref[...]) + bt_ref[...]
    e_t_feats = _mm_t(feats, wt_ref[...]) + bt_ref[...]
    eh_ref[0] = e_h
    def pack_bf16(x):
        bits = jax.lax.bitcast_convert_type(x, jnp.int32)
        rnd = bits + jnp.int32(0x7FFF) + (
            jax.lax.shift_right_logical(bits, 16) & jnp.int32(1))
        left = rnd[:, : d // 2]
        right = rnd[:, d // 2:]
        lo = jax.lax.shift_right_logical(left, 16)
        hi = right & jnp.int32(np.int32(np.uint32(0xFFFF0000)))
        return hi | lo
    et_ref[0, pl.ds(0, h_dim), :] = pack_bf16(e_t_cls)
    et_ref[0, pl.ds(h_dim, t_dim), :] = pack_bf16(e_t_feats)

    e_hs = e_h * scale
    attn = jnp.concatenate(
        [_mm_t(e_hs, e_t_cls), _mm_t(e_hs, e_t_feats)], axis=1)

    bits = jax.lax.bitcast_convert_type(attn, jnp.int32)
    skey = bits ^ (jax.lax.shift_right_arithmetic(bits, 31)
                   & jnp.int32(0x7FFFFFFF))
    sign = jnp.int32(np.int32(np.uint32(0x80000000)))
    prefix_u = jnp.zeros((h_dim, 1), dtype=jnp.int32)
    for b in range(31, -1, -1):
        bitc = jnp.int32(np.int32(np.uint32(1 << b)))
        cand_u = prefix_u | bitc
        cand_s = cand_u ^ sign
        cnt = jnp.sum((skey >= cand_s).astype(jnp.int32), axis=1,
                      keepdims=True)
        prefix_u = jnp.where(cnt >= k, cand_u, prefix_u)
    thr_s = prefix_u ^ sign

    mask_gt = skey > thr_s
    mask_eq = skey == thr_s
    cnt_gt = jnp.sum(mask_gt.astype(jnp.float32), axis=1, keepdims=True)
    need = jnp.float32(k) - cnt_gt
    rank_eq = _lane_cumsum(mask_eq.astype(jnp.float32))
    mask = mask_gt | (mask_eq & (rank_eq <= need))
    rank = _lane_cumsum(mask.astype(jnp.int32))
    rank_sel = jnp.where(mask, rank - 1, -1)

    n = h_dim + t_dim
    for blk in range(h_dim // H_BLK):
        h0 = blk * H_BLK
        rk3 = jnp.broadcast_to(rank_sel[h0:h0 + H_BLK, :][:, None, :],
                               (H_BLK, k, n))
        s_iota = jax.lax.broadcasted_iota(jnp.int32, (H_BLK, k, n), 1)
        j_iota = jax.lax.broadcasted_iota(jnp.int32, (H_BLK, k, n), 2)
        idx_blk = jnp.sum(jnp.where(rk3 == s_iota, j_iota, 0), axis=2)
        idx_ref[0, pl.ds(h0, H_BLK), :] = idx_blk


def _sc_gather(idx_hbm, table_hbm, nb_hbm, idx_v, rows_v, sem, *, ipw, chunk):
    wid = lax.axis_index("s") * SC_NC + lax.axis_index("c")
    base = wid * ipw
    pltpu.sync_copy(idx_hbm.at[pl.ds(base, ipw)], idx_v)

    def g_body(c, carry):
        pltpu.async_copy(table_hbm.at[idx_v.at[pl.ds(c * chunk, chunk)]],
                         rows_v, sem).wait()
        pltpu.sync_copy(rows_v, nb_hbm.at[pl.ds(base + c * chunk, chunk)])
        return carry
    lax.fori_loop(0, ipw // chunk, g_body, 0)


def _stage_c(cls_ref, eh_ref, nb_ref, w1_ref, b1_ref, w2_ref, b2_ref,
             gamma_ref, beta_ref, out_ref, enh_ref, *, k, n_blk):
    blk = pl.program_id(1)
    h_dim, d = cls_ref.shape[1:]
    scale = d ** (-0.5)

    h0 = blk * H_BLK
    packed = nb_ref[0, 0]
    lo_f = jax.lax.bitcast_convert_type(
        jax.lax.shift_left(packed, 16), jnp.float32)
    hi_f = jax.lax.bitcast_convert_type(
        packed & jnp.int32(np.int32(np.uint32(0xFFFF0000))),
        jnp.float32)
    nb3 = jnp.concatenate([lo_f, hi_f], axis=1).reshape(H_BLK, k, d)
    e_h_blk = eh_ref[0, pl.ds(h0, H_BLK), :]
    e_hs_b = (e_h_blk * scale)[:, None, :]
    sel_logit = jnp.sum(nb3 * e_hs_b, axis=2, keepdims=True)
    m = jnp.max(sel_logit, axis=1, keepdims=True)
    ex = jnp.exp(sel_logit - m)
    pk = ex / jnp.sum(ex, axis=1, keepdims=True)

    arg = (2.0 - pk) * e_h_blk[:, None, :] + pk * nb3
    gate = jnp.tanh(arg)
    kaw = jnp.sum(nb3 * gate, axis=2, keepdims=True)
    km = jnp.max(kaw, axis=1, keepdims=True)
    kex = jnp.exp(kaw - km)
    ka_prob = kex / jnp.sum(kex, axis=1, keepdims=True)
    e_nh = jnp.sum(ka_prob * nb3, axis=1)
    enh_ref[pl.ds(h0, H_BLK), :] = e_nh

    @pl.when(blk == n_blk - 1)
    def _tail():
        cls = cls_ref[0]
        e_h = eh_ref[0]
        e_nh_all = enh_ref[...]
        sum_in = (e_h + e_nh_all) * 0.1 + cls
        bi_in = e_h * e_nh_all * 0.1 + cls
        s_emb = _leaky(_mm_t(sum_in, w1_ref[...]) + b1_ref[...])
        b_emb = _leaky(_mm_t(bi_in, w2_ref[...]) + b2_ref[...])
        emb = s_emb + b_emb
        mu = jnp.mean(emb, axis=-1, keepdims=True)
        var = jnp.mean((emb - mu) ** 2, axis=-1, keepdims=True)
        out = (emb - mu) / jnp.sqrt(var + 1e-5) * gamma_ref[...] + beta_ref[...]
        out_ref[0] = out


@jax.jit
def kernel(cls_tokens, feats, Wh, bh, Wt, bt, W1, b1, W2, b2, gamma, beta):
    b_dim, h_dim, d = cls_tokens.shape
    t_dim = feats.shape[1]
    n = h_dim + t_dim
    k = max(1, min(t_dim, int(0.5 * max(1, h_dim))))

    row = lambda v: v.reshape(1, d)
    full = lambda shape: pl.BlockSpec(shape, lambda *g: (0,) * len(shape))

    n_blk = h_dim // H_BLK
    n_rows = h_dim * k
    ipw = n_rows // (SC_NC * SC_NS)
    mesh = plsc.VectorSubcoreMesh(core_axis_name="c", subcore_axis_name="s")

    stage_a = pl.pallas_call(
        functools.partial(_stage_a, k=k),
        grid=(1,),
        in_specs=[
            pl.BlockSpec((1, h_dim, d), lambda b: (b, 0, 0)),
            pl.BlockSpec((1, t_dim, d), lambda b: (b, 0, 0)),
            full((d, d)), full((1, d)),
            full((d, d)), full((1, d)),
        ],
        out_specs=[
            pl.BlockSpec((1, h_dim, d), lambda b: (b, 0, 0)),
            pl.BlockSpec((1, n, d // 2), lambda b: (b, 0, 0)),
            pl.BlockSpec((1, h_dim, k), lambda b: (b, 0, 0)),
        ],
        out_shape=[
            jax.ShapeDtypeStruct((1, h_dim, d), jnp.float32),
            jax.ShapeDtypeStruct((1, n, d // 2), jnp.int32),
            jax.ShapeDtypeStruct((1, h_dim, k), jnp.int32),
        ],
    )

    sc_gather = pl.kernel(
        functools.partial(_sc_gather, ipw=ipw, chunk=SC_CHUNK),
        mesh=mesh,
        out_type=jax.ShapeDtypeStruct((n_rows, d // 2), jnp.int32),
        scratch_types=[
            pltpu.VMEM((ipw,), jnp.int32),
            pltpu.VMEM((SC_CHUNK, d // 2), jnp.int32),
            pltpu.SemaphoreType.DMA,
        ],
    )

    stage_c = pl.pallas_call(
        functools.partial(_stage_c, k=k, n_blk=n_blk),
        grid=(1, n_blk),
        in_specs=[
            pl.BlockSpec((1, h_dim, d), lambda b, j: (b, 0, 0)),
            pl.BlockSpec((1, h_dim, d), lambda b, j: (b, 0, 0)),
            pl.BlockSpec((1, 1, H_BLK * k, d // 2), lambda b, j: (b, j, 0, 0)),
            full((d, d)), full((1, d)),
            full((d, d)), full((1, d)),
            full((1, d)), full((1, d)),
        ],
        out_specs=pl.BlockSpec((1, h_dim, d), lambda b, j: (b, 0, 0)),
        out_shape=jax.ShapeDtypeStruct((1, h_dim, d), jnp.float32),
        scratch_shapes=[
            pltpu.VMEM((h_dim, d), jnp.float32),
        ],
    )

    outs = []
    for b in range(b_dim):
        eh_b, et_b, idx_b = stage_a(
            cls_tokens[b:b + 1], feats[b:b + 1], Wh, row(bh), Wt, row(bt))
        nb_b = sc_gather(idx_b.reshape(n_rows), et_b.reshape(n, d // 2))
        outs.append(stage_c(
            cls_tokens[b:b + 1], eh_b,
            nb_b.reshape(1, n_blk, H_BLK * k, d // 2),
            W1, row(b1), W2, row(b2), row(gamma), row(beta)))
    return jnp.concatenate(outs, axis=0)
